# baseline (speedup 1.0000x reference)
import functools

import jax
import jax.numpy as jnp
from jax import lax
from jax.experimental import pallas as pl
from jax.experimental.pallas import tpu as pltpu
from jax.experimental.pallas import tpu_sc as plsc

_BATCH = 16384
_EMBED = 16
_HOUR_VOCAB = 24
_PHONE_VOCAB = 100000
_NC = 2
_NS = 16
_NW = _NC * _NS
_B_PER_W = _BATCH // _NW
_G = 16
_CHUNK = 64
_STRIP = 256


@jax.jit
def _context_embedding_sc(hour_idx, phone_idx, hour_table, phone_table):
    mesh = plsc.VectorSubcoreMesh(core_axis_name="c", subcore_axis_name="s")

    @functools.partial(
        pl.kernel,
        mesh=mesh,
        out_type=jax.ShapeDtypeStruct((_BATCH, 2 * _EMBED), jnp.float32),
        scratch_types=[
            pltpu.VMEM((_HOUR_VOCAB, _EMBED), jnp.float32),
            pltpu.VMEM((_B_PER_W,), jnp.int32),
            pltpu.VMEM((_B_PER_W,), jnp.int32),
            pltpu.VMEM((_CHUNK, 8, _EMBED), jnp.float32),
            pltpu.VMEM((_STRIP, 2 * _EMBED), jnp.float32),
            pltpu.SemaphoreType.DMA,
        ],
    )
    def k(hi_hbm, pi_hbm, ht_hbm, pt_hbm, out_hbm,
          ht_v, hi_v, pi_v, blk_v, cat_v, sem):
        wid = lax.axis_index("s") * _NC + lax.axis_index("c")
        base = wid * _B_PER_W
        pltpu.sync_copy(hi_hbm.at[pl.ds(base, _B_PER_W)], hi_v)
        pltpu.sync_copy(pi_hbm.at[pl.ds(base, _B_PER_W)], pi_v)
        pltpu.sync_copy(ht_hbm, ht_v)

        for c in range(_B_PER_W // _CHUNK):
            @pl.loop(0, _CHUNK // _G)
            def _(g):
                pvec = pi_v[pl.ds(c * _CHUNK + g * _G, _G)] & ~7
                for j in range(_G):
                    off = pl.multiple_of(pvec[j], 8)
                    pltpu.async_copy(
                        pt_hbm.at[pl.ds(off, 8)],
                        blk_v.at[g * _G + j],
                        sem)

            @pl.loop(0, _CHUNK)
            def _(d):
                pltpu.make_async_copy(
                    pt_hbm.at[pl.ds(0, 8)], blk_v.at[0], sem).wait()

            @pl.loop(0, _CHUNK // _G)
            def _(g):
                e0 = c * _CHUNK + g * _G
                hvec = hi_v[pl.ds(e0, _G)]
                pvec = pi_v[pl.ds(e0, _G)] & 7
                for j in range(_G):
                    i = (c * _CHUNK) % _STRIP + g * _G + j
                    cat_v.at[i, pl.ds(0, _EMBED)][...] = (
                        ht_v.at[hvec[j], pl.ds(0, _EMBED)][...])
                    cat_v.at[i, pl.ds(_EMBED, _EMBED)][...] = (
                        blk_v.at[g * _G + j, pvec[j], pl.ds(0, _EMBED)][...])

            if (c + 1) % (_STRIP // _CHUNK) == 0:
                s = (c + 1) * _CHUNK - _STRIP
                pltpu.sync_copy(cat_v, out_hbm.at[pl.ds(base + s, _STRIP)])

    return k(hour_idx, phone_idx, hour_table, phone_table)


def kernel(hour_idx, phone_idx, hour_table, phone_table):
    return _context_embedding_sc(
        hour_idx.astype(jnp.int32),
        phone_idx.astype(jnp.int32),
        hour_table,
        phone_table,
    )
